# baseline (speedup 1.0000x reference)
import numpy as np
import jax
import jax.numpy as jnp
from jax import lax
from jax.experimental import pallas as pl
from jax.experimental.pallas import tpu as pltpu
from jax.experimental.pallas import tpu_sc as plsc

HIDDEN = 768
NUM_HEADS = 12
NUM_KV_HEADS = 4
HEAD_DIM = 64
HALF = HEAD_DIM // 2
NUM_EXPERTS = 64
TOP_K = 8
ROPE_THETA = 10000.0
T = 2048
Q_SIZE = NUM_HEADS * HEAD_DIM
KV_SIZE = NUM_KV_HEADS * HEAD_DIM
SCALING = HEAD_DIM ** -0.5
EPS = 1e-6
REP = NUM_HEADS // NUM_KV_HEADS

TB = 256
QB = 512
KC = 512

_INV_FREQ = (1.0 / (ROPE_THETA ** (np.arange(0, HALF, dtype=np.float32) * 2.0 / HEAD_DIM))).astype(np.float32)

SC_NC = 2
SC_NS = 16
NW = SC_NC * SC_NS
N_PERM = T * TOP_K
ROWS_PER_W = N_PERM // NW
GCHUNK = 64


def _qkv_body(x_ref, wpre_ref, wqkv_ref, q_ref, k_ref, v_ref):
    x = x_ref[...]
    var = jnp.mean(x * x, axis=-1, keepdims=True)
    h = x * lax.rsqrt(var + EPS) * wpre_ref[...]
    qkv = lax.dot_general(h, wqkv_ref[...], (((1,), (1,)), ((), ())),
                          preferred_element_type=jnp.float32)
    q_ref[...] = qkv[:, :Q_SIZE]
    k_ref[...] = qkv[:, Q_SIZE:Q_SIZE + KV_SIZE]
    v_ref[...] = qkv[:, Q_SIZE + KV_SIZE:]


def _rope(x, pos):
    expo = lax.broadcasted_iota(jnp.int32, (1, HALF), 1).astype(jnp.float32) * (2.0 / HEAD_DIM)
    inv_freq = jnp.exp(expo * (-float(np.log(ROPE_THETA))))
    freqs = pos * inv_freq
    c = jnp.cos(freqs)
    s = jnp.sin(freqs)
    x1 = x[:, :HALF]
    x2 = x[:, HALF:]
    return jnp.concatenate([x1 * c - x2 * s, x2 * c + x1 * s], axis=1)


def _attn_body(posq_ref, posk_ref, q_ref, k_ref, v_ref, o_ref):
    qb = pl.program_id(0)
    posq = posq_ref[...]
    qs = q_ref[...]
    row = qb * QB + lax.broadcasted_iota(jnp.int32, (QB, KC), 0)
    coli = lax.broadcasted_iota(jnp.int32, (QB, KC), 1)
    for h in range(NUM_HEADS):
        kh = h // REP
        q = _rope(qs[:, h * HEAD_DIM:(h + 1) * HEAD_DIM], posq) * SCALING

        def chunk(ci, carry):
            m, l, acc = carry
            ks = k_ref[pl.ds(ci * KC, KC), kh * HEAD_DIM:(kh + 1) * HEAD_DIM]
            vs = v_ref[pl.ds(ci * KC, KC), kh * HEAD_DIM:(kh + 1) * HEAD_DIM]
            posk = posk_ref[pl.ds(ci * KC, KC), :]
            k = _rope(ks, posk)
            s = lax.dot_general(q, k, (((1,), (1,)), ((), ())),
                                preferred_element_type=jnp.float32)
            s = jnp.where(row >= ci * KC + coli, s, jnp.finfo(jnp.float32).min)
            m_new = jnp.maximum(m, jnp.max(s, axis=-1, keepdims=True))
            corr = jnp.exp(m - m_new)
            p = jnp.exp(s - m_new)
            l_new = l * corr + jnp.sum(p, axis=-1, keepdims=True)
            pv = lax.dot_general(p, vs, (((1,), (0,)), ((), ())),
                                 preferred_element_type=jnp.float32)
            acc_new = acc * corr + pv
            return m_new, l_new, acc_new

        m0 = jnp.full((QB, 1), -jnp.inf, dtype=jnp.float32)
        l0 = jnp.zeros((QB, 1), dtype=jnp.float32)
        a0 = jnp.zeros((QB, HEAD_DIM), dtype=jnp.float32)
        m, l, acc = lax.fori_loop(0, qb + 1, chunk, (m0, l0, a0))
        o_ref[:, h * HEAD_DIM:(h + 1) * HEAD_DIM] = acc / l


def _oproj_body(a_ref, wo_ref, res_ref, wpost_ref, out_ref):
    o = lax.dot_general(a_ref[...], wo_ref[...], (((1,), (1,)), ((), ())),
                        preferred_element_type=jnp.float32)
    r = o + res_ref[...]
    var = jnp.mean(r * r, axis=-1, keepdims=True)
    out_ref[...] = r * lax.rsqrt(var + EPS) * wpost_ref[...]


def _sc_gather_body(table_hbm, idx_hbm, out_hbm, idx_v, buf0, buf1, sem0, sem1):
    wid = lax.axis_index("s") * SC_NC + lax.axis_index("c")
    base = wid * ROWS_PER_W
    pltpu.sync_copy(idx_hbm.at[pl.ds(base, ROWS_PER_W)], idx_v)
    nchunk = ROWS_PER_W // GCHUNK
    bufs = (buf0, buf1)
    sems = (sem0, sem1)
    handles = [None, None]
    handles[0] = pltpu.async_copy(
        table_hbm.at[idx_v.at[pl.ds(0, GCHUNK)]], bufs[0], sems[0])
    for c in range(1, nchunk + 1):
        if c < nchunk:
            b = c % 2
            handles[b] = pltpu.async_copy(
                table_hbm.at[idx_v.at[pl.ds(c * GCHUNK, GCHUNK)]], bufs[b], sems[b])
        pb = (c - 1) % 2
        handles[pb].wait()
        pltpu.sync_copy(bufs[pb], out_hbm.at[pl.ds(base + (c - 1) * GCHUNK, GCHUNK)])


def _permute_gather_sc(out, src_idx):
    mesh = plsc.VectorSubcoreMesh(core_axis_name="c", subcore_axis_name="s")
    kfn = pl.kernel(
        _sc_gather_body,
        out_type=jax.ShapeDtypeStruct((N_PERM, HIDDEN), jnp.float32),
        mesh=mesh,
        scratch_types=[
            pltpu.VMEM((ROWS_PER_W,), jnp.int32),
            pltpu.VMEM((GCHUNK, HIDDEN), jnp.float32),
            pltpu.VMEM((GCHUNK, HIDDEN), jnp.float32),
            pltpu.SemaphoreType.DMA,
            pltpu.SemaphoreType.DMA,
        ],
    )
    return kfn(out, src_idx)


def kernel(positions, hidden_states, kv_cache, w_pre, W_qkv, W_o, w_post, W_gate):
    pos2d = positions.astype(jnp.float32).reshape(T, 1)
    wpre2d = w_pre.reshape(1, HIDDEN)
    wpost2d = w_post.reshape(1, HIDDEN)

    q, k, v = pl.pallas_call(
        _qkv_body,
        grid=(T // TB,),
        in_specs=[
            pl.BlockSpec((TB, HIDDEN), lambda i: (i, 0)),
            pl.BlockSpec((1, HIDDEN), lambda i: (0, 0)),
            pl.BlockSpec((Q_SIZE + 2 * KV_SIZE, HIDDEN), lambda i: (0, 0)),
        ],
        out_specs=[
            pl.BlockSpec((TB, Q_SIZE), lambda i: (i, 0)),
            pl.BlockSpec((TB, KV_SIZE), lambda i: (i, 0)),
            pl.BlockSpec((TB, KV_SIZE), lambda i: (i, 0)),
        ],
        out_shape=[
            jax.ShapeDtypeStruct((T, Q_SIZE), jnp.float32),
            jax.ShapeDtypeStruct((T, KV_SIZE), jnp.float32),
            jax.ShapeDtypeStruct((T, KV_SIZE), jnp.float32),
        ],
    )(hidden_states, wpre2d, W_qkv)

    attn = pl.pallas_call(
        _attn_body,
        grid=(T // QB,),
        in_specs=[
            pl.BlockSpec((QB, 1), lambda qb: (qb, 0)),
            pl.BlockSpec((T, 1), lambda qb: (0, 0)),
            pl.BlockSpec((QB, Q_SIZE), lambda qb: (qb, 0)),
            pl.BlockSpec((T, KV_SIZE), lambda qb: (0, 0)),
            pl.BlockSpec((T, KV_SIZE), lambda qb: (0, 0)),
        ],
        out_specs=pl.BlockSpec((QB, Q_SIZE), lambda qb: (qb, 0)),
        out_shape=jax.ShapeDtypeStruct((T, Q_SIZE), jnp.float32),
    )(pos2d, pos2d, q, k, v)

    out = pl.pallas_call(
        _oproj_body,
        grid=(T // TB,),
        in_specs=[
            pl.BlockSpec((TB, Q_SIZE), lambda i: (i, 0)),
            pl.BlockSpec((HIDDEN, Q_SIZE), lambda i: (0, 0)),
            pl.BlockSpec((TB, HIDDEN), lambda i: (i, 0)),
            pl.BlockSpec((1, HIDDEN), lambda i: (0, 0)),
        ],
        out_specs=pl.BlockSpec((TB, HIDDEN), lambda i: (i, 0)),
        out_shape=jax.ShapeDtypeStruct((T, HIDDEN), jnp.float32),
    )(attn, W_o, hidden_states, wpost2d)

    key_r = jax.random.fold_in(jax.random.key(0), 123)
    expert_logits = jax.random.uniform(key_r, (T, NUM_EXPERTS), dtype=jnp.float32)
    topk_weights, topk_ids = lax.top_k(expert_logits, TOP_K)
    topk_weights = topk_weights / jnp.sum(topk_weights, axis=-1, keepdims=True)
    flat_ids = topk_ids.reshape(-1)
    reorder_ids = jnp.argsort(flat_ids)
    src_idx = (reorder_ids // TOP_K).astype(jnp.int32)

    permuted_output = _permute_gather_sc(out, src_idx)
    return (permuted_output, topk_weights, topk_ids, reorder_ids)


# speedup vs baseline: 1.4997x; 1.4997x over previous
import numpy as np
import jax
import jax.numpy as jnp
from jax import lax
from jax.experimental import pallas as pl
from jax.experimental.pallas import tpu as pltpu
from jax.experimental.pallas import tpu_sc as plsc

HIDDEN = 768
NUM_HEADS = 12
NUM_KV_HEADS = 4
HEAD_DIM = 64
HALF = HEAD_DIM // 2
NUM_EXPERTS = 64
TOP_K = 8
ROPE_THETA = 10000.0
T = 2048
Q_SIZE = NUM_HEADS * HEAD_DIM
KV_SIZE = NUM_KV_HEADS * HEAD_DIM
SCALING = HEAD_DIM ** -0.5
EPS = 1e-6
REP = NUM_HEADS // NUM_KV_HEADS

TB = 256
QB = 256

_INV_FREQ = (1.0 / (ROPE_THETA ** (np.arange(0, HALF, dtype=np.float32) * 2.0 / HEAD_DIM))).astype(np.float32)

SC_NC = 2
SC_NS = 16
NW = SC_NC * SC_NS
N_PERM = T * TOP_K
ROWS_PER_W = N_PERM // NW
GCHUNK = 64


def _qkv_body(x_ref, wpre_ref, wqkv_ref, q_ref, k_ref, v_ref):
    x = x_ref[...]
    var = jnp.mean(x * x, axis=-1, keepdims=True)
    h = x * lax.rsqrt(var + EPS) * wpre_ref[...]
    qkv = lax.dot_general(h, wqkv_ref[...], (((1,), (1,)), ((), ())),
                          preferred_element_type=jnp.float32)
    q_ref[...] = qkv[:, :Q_SIZE]
    k_ref[...] = qkv[:, Q_SIZE:Q_SIZE + KV_SIZE]
    v_ref[...] = qkv[:, Q_SIZE + KV_SIZE:]


def _rope(x, pos):
    expo = lax.broadcasted_iota(jnp.int32, (1, HALF), 1).astype(jnp.float32) * (2.0 / HEAD_DIM)
    inv_freq = jnp.exp(expo * (-float(np.log(ROPE_THETA))))
    freqs = pos * inv_freq
    c = jnp.cos(freqs)
    s = jnp.sin(freqs)
    x1 = x[:, :HALF]
    x2 = x[:, HALF:]
    return jnp.concatenate([x1 * c - x2 * s, x2 * c + x1 * s], axis=1)


def _attn_body(posq_ref, posk_ref, q_ref, k_ref, v_ref, o_ref):
    qb = pl.program_id(0)
    posq = posq_ref[...]
    posk = posk_ref[...]
    row = qb * QB + lax.broadcasted_iota(jnp.int32, (QB, T), 0)
    col = lax.broadcasted_iota(jnp.int32, (QB, T), 1)
    causal = row >= col
    qs = q_ref[...]
    ks = k_ref[...]
    vs = v_ref[...]
    kro = [None] * NUM_KV_HEADS
    for kh in range(NUM_KV_HEADS):
        kro[kh] = _rope(ks[:, kh * HEAD_DIM:(kh + 1) * HEAD_DIM], posk).astype(jnp.bfloat16)
    for h in range(NUM_HEADS):
        kh = h // REP
        q = (_rope(qs[:, h * HEAD_DIM:(h + 1) * HEAD_DIM], posq) * SCALING).astype(jnp.bfloat16)
        v = vs[:, kh * HEAD_DIM:(kh + 1) * HEAD_DIM].astype(jnp.bfloat16)
        s = lax.dot_general(q, kro[kh], (((1,), (1,)), ((), ())),
                            preferred_element_type=jnp.float32)
        s = jnp.where(causal, s, jnp.finfo(jnp.float32).min)
        m = jnp.max(s, axis=-1, keepdims=True)
        p = jnp.exp(s - m)
        l = jnp.sum(p, axis=-1, keepdims=True)
        o = lax.dot_general(p.astype(jnp.bfloat16), v, (((1,), (0,)), ((), ())),
                            preferred_element_type=jnp.float32)
        o_ref[:, h * HEAD_DIM:(h + 1) * HEAD_DIM] = o / l


def _oproj_body(a_ref, wo_ref, res_ref, wpost_ref, out_ref):
    o = lax.dot_general(a_ref[...], wo_ref[...], (((1,), (1,)), ((), ())),
                        preferred_element_type=jnp.float32)
    r = o + res_ref[...]
    var = jnp.mean(r * r, axis=-1, keepdims=True)
    out_ref[...] = r * lax.rsqrt(var + EPS) * wpost_ref[...]


def _sc_gather_body(table_hbm, idx_hbm, out_hbm, idx_v, buf0, buf1, sem0, sem1):
    wid = lax.axis_index("s") * SC_NC + lax.axis_index("c")
    base = wid * ROWS_PER_W
    pltpu.sync_copy(idx_hbm.at[pl.ds(base, ROWS_PER_W)], idx_v)
    nchunk = ROWS_PER_W // GCHUNK
    bufs = (buf0, buf1)
    sems = (sem0, sem1)
    handles = [None, None]
    handles[0] = pltpu.async_copy(
        table_hbm.at[idx_v.at[pl.ds(0, GCHUNK)]], bufs[0], sems[0])
    for c in range(1, nchunk + 1):
        if c < nchunk:
            b = c % 2
            handles[b] = pltpu.async_copy(
                table_hbm.at[idx_v.at[pl.ds(c * GCHUNK, GCHUNK)]], bufs[b], sems[b])
        pb = (c - 1) % 2
        handles[pb].wait()
        pltpu.sync_copy(bufs[pb], out_hbm.at[pl.ds(base + (c - 1) * GCHUNK, GCHUNK)])


def _permute_gather_sc(out, src_idx):
    mesh = plsc.VectorSubcoreMesh(core_axis_name="c", subcore_axis_name="s")
    kfn = pl.kernel(
        _sc_gather_body,
        out_type=jax.ShapeDtypeStruct((N_PERM, HIDDEN), jnp.float32),
        mesh=mesh,
        scratch_types=[
            pltpu.VMEM((ROWS_PER_W,), jnp.int32),
            pltpu.VMEM((GCHUNK, HIDDEN), jnp.float32),
            pltpu.VMEM((GCHUNK, HIDDEN), jnp.float32),
            pltpu.SemaphoreType.DMA,
            pltpu.SemaphoreType.DMA,
        ],
    )
    return kfn(out, src_idx)


def kernel(positions, hidden_states, kv_cache, w_pre, W_qkv, W_o, w_post, W_gate):
    pos2d = positions.astype(jnp.float32).reshape(T, 1)
    wpre2d = w_pre.reshape(1, HIDDEN)
    wpost2d = w_post.reshape(1, HIDDEN)

    q, k, v = pl.pallas_call(
        _qkv_body,
        grid=(T // TB,),
        in_specs=[
            pl.BlockSpec((TB, HIDDEN), lambda i: (i, 0)),
            pl.BlockSpec((1, HIDDEN), lambda i: (0, 0)),
            pl.BlockSpec((Q_SIZE + 2 * KV_SIZE, HIDDEN), lambda i: (0, 0)),
        ],
        out_specs=[
            pl.BlockSpec((TB, Q_SIZE), lambda i: (i, 0)),
            pl.BlockSpec((TB, KV_SIZE), lambda i: (i, 0)),
            pl.BlockSpec((TB, KV_SIZE), lambda i: (i, 0)),
        ],
        out_shape=[
            jax.ShapeDtypeStruct((T, Q_SIZE), jnp.float32),
            jax.ShapeDtypeStruct((T, KV_SIZE), jnp.float32),
            jax.ShapeDtypeStruct((T, KV_SIZE), jnp.float32),
        ],
    )(hidden_states, wpre2d, W_qkv)

    attn = pl.pallas_call(
        _attn_body,
        grid=(T // QB,),
        in_specs=[
            pl.BlockSpec((QB, 1), lambda qb: (qb, 0)),
            pl.BlockSpec((T, 1), lambda qb: (0, 0)),
            pl.BlockSpec((QB, Q_SIZE), lambda qb: (qb, 0)),
            pl.BlockSpec((T, KV_SIZE), lambda qb: (0, 0)),
            pl.BlockSpec((T, KV_SIZE), lambda qb: (0, 0)),
        ],
        out_specs=pl.BlockSpec((QB, Q_SIZE), lambda qb: (qb, 0)),
        out_shape=jax.ShapeDtypeStruct((T, Q_SIZE), jnp.float32),
    )(pos2d, pos2d, q, k, v)

    out = pl.pallas_call(
        _oproj_body,
        grid=(T // TB,),
        in_specs=[
            pl.BlockSpec((TB, Q_SIZE), lambda i: (i, 0)),
            pl.BlockSpec((HIDDEN, Q_SIZE), lambda i: (0, 0)),
            pl.BlockSpec((TB, HIDDEN), lambda i: (i, 0)),
            pl.BlockSpec((1, HIDDEN), lambda i: (0, 0)),
        ],
        out_specs=pl.BlockSpec((TB, HIDDEN), lambda i: (i, 0)),
        out_shape=jax.ShapeDtypeStruct((T, HIDDEN), jnp.float32),
    )(attn, W_o, hidden_states, wpost2d)

    key_r = jax.random.fold_in(jax.random.key(0), 123)
    expert_logits = jax.random.uniform(key_r, (T, NUM_EXPERTS), dtype=jnp.float32)
    topk_weights, topk_ids = lax.top_k(expert_logits, TOP_K)
    topk_weights = topk_weights / jnp.sum(topk_weights, axis=-1, keepdims=True)
    flat_ids = topk_ids.reshape(-1)
    reorder_ids = jnp.argsort(flat_ids)
    src_idx = (reorder_ids // TOP_K).astype(jnp.int32)

    permuted_output = _permute_gather_sc(out, src_idx)
    return (permuted_output, topk_weights, topk_ids, reorder_ids)


# speedup vs baseline: 1.8214x; 1.2145x over previous
import numpy as np
import jax
import jax.numpy as jnp
from jax import lax
from jax.experimental import pallas as pl
from jax.experimental.pallas import tpu as pltpu
from jax.experimental.pallas import tpu_sc as plsc

HIDDEN = 768
NUM_HEADS = 12
NUM_KV_HEADS = 4
HEAD_DIM = 64
HALF = HEAD_DIM // 2
NUM_EXPERTS = 64
TOP_K = 8
ROPE_THETA = 10000.0
T = 2048
Q_SIZE = NUM_HEADS * HEAD_DIM
KV_SIZE = NUM_KV_HEADS * HEAD_DIM
SCALING = HEAD_DIM ** -0.5
EPS = 1e-6
REP = NUM_HEADS // NUM_KV_HEADS

TB = 256
QB = 256

SC_NC = 2
SC_NS = 16
NW = SC_NC * SC_NS
N_PERM = T * TOP_K
ROWS_PER_W = N_PERM // NW
GCHUNK = 64


def _qkv_body(pos_ref, x_ref, wpre_ref, wqkv_ref, q_ref, k_ref, v_ref):
    x = x_ref[...]
    var = jnp.mean(x * x, axis=-1, keepdims=True)
    h = x * lax.rsqrt(var + EPS) * wpre_ref[...]
    qkv = lax.dot_general(h, wqkv_ref[...], (((1,), (1,)), ((), ())),
                          preferred_element_type=jnp.float32)
    pos = pos_ref[...]
    expo = lax.broadcasted_iota(jnp.int32, (1, HALF), 1).astype(jnp.float32) * (2.0 / HEAD_DIM)
    inv_freq = jnp.exp(expo * (-float(np.log(ROPE_THETA))))
    freqs = pos * inv_freq
    c = jnp.cos(freqs)
    s = jnp.sin(freqs)

    def rot(x):
        x1 = x[:, :HALF]
        x2 = x[:, HALF:]
        return jnp.concatenate([x1 * c - x2 * s, x2 * c + x1 * s], axis=1)

    for hh in range(NUM_HEADS):
        sl = slice(hh * HEAD_DIM, (hh + 1) * HEAD_DIM)
        q_ref[:, sl] = (rot(qkv[:, sl]) * SCALING).astype(jnp.bfloat16)
    for kh in range(NUM_KV_HEADS):
        sl = slice(Q_SIZE + kh * HEAD_DIM, Q_SIZE + (kh + 1) * HEAD_DIM)
        k_ref[:, kh * HEAD_DIM:(kh + 1) * HEAD_DIM] = rot(qkv[:, sl]).astype(jnp.bfloat16)
    v_ref[...] = qkv[:, Q_SIZE + KV_SIZE:].astype(jnp.bfloat16)


def _attn_body(q_ref, k_ref, v_ref, o_ref):
    qb = pl.program_id(0)
    row = qb * QB + lax.broadcasted_iota(jnp.int32, (QB, T), 0)
    col = lax.broadcasted_iota(jnp.int32, (QB, T), 1)
    causal = row >= col
    qs = q_ref[...]
    ks = k_ref[...]
    vs = v_ref[...]
    for h in range(NUM_HEADS):
        kh = h // REP
        q = qs[:, h * HEAD_DIM:(h + 1) * HEAD_DIM]
        k = ks[:, kh * HEAD_DIM:(kh + 1) * HEAD_DIM]
        v = vs[:, kh * HEAD_DIM:(kh + 1) * HEAD_DIM]
        s = lax.dot_general(q, k, (((1,), (1,)), ((), ())),
                            preferred_element_type=jnp.float32)
        s = jnp.where(causal, s, jnp.finfo(jnp.float32).min)
        m = jnp.max(s, axis=-1, keepdims=True)
        p = jnp.exp(s - m)
        l = jnp.sum(p, axis=-1, keepdims=True)
        o = lax.dot_general(p.astype(jnp.bfloat16), v, (((1,), (0,)), ((), ())),
                            preferred_element_type=jnp.float32)
        o_ref[:, h * HEAD_DIM:(h + 1) * HEAD_DIM] = o / l


def _oproj_body(a_ref, wo_ref, res_ref, wpost_ref, out_ref):
    o = lax.dot_general(a_ref[...], wo_ref[...], (((1,), (1,)), ((), ())),
                        preferred_element_type=jnp.float32)
    r = o + res_ref[...]
    var = jnp.mean(r * r, axis=-1, keepdims=True)
    out_ref[...] = r * lax.rsqrt(var + EPS) * wpost_ref[...]


def _sc_gather_body(table_hbm, idx_hbm, out_hbm, idx_v, buf0, buf1, sem0, sem1):
    wid = lax.axis_index("s") * SC_NC + lax.axis_index("c")
    base = wid * ROWS_PER_W
    pltpu.sync_copy(idx_hbm.at[pl.ds(base, ROWS_PER_W)], idx_v)
    nchunk = ROWS_PER_W // GCHUNK
    bufs = (buf0, buf1)
    sems = (sem0, sem1)
    handles = [None, None]
    handles[0] = pltpu.async_copy(
        table_hbm.at[idx_v.at[pl.ds(0, GCHUNK)]], bufs[0], sems[0])
    for c in range(1, nchunk + 1):
        if c < nchunk:
            b = c % 2
            handles[b] = pltpu.async_copy(
                table_hbm.at[idx_v.at[pl.ds(c * GCHUNK, GCHUNK)]], bufs[b], sems[b])
        pb = (c - 1) % 2
        handles[pb].wait()
        pltpu.sync_copy(bufs[pb], out_hbm.at[pl.ds(base + (c - 1) * GCHUNK, GCHUNK)])


def _permute_gather_sc(out, src_idx):
    mesh = plsc.VectorSubcoreMesh(core_axis_name="c", subcore_axis_name="s")
    kfn = pl.kernel(
        _sc_gather_body,
        out_type=jax.ShapeDtypeStruct((N_PERM, HIDDEN), jnp.float32),
        mesh=mesh,
        scratch_types=[
            pltpu.VMEM((ROWS_PER_W,), jnp.int32),
            pltpu.VMEM((GCHUNK, HIDDEN), jnp.float32),
            pltpu.VMEM((GCHUNK, HIDDEN), jnp.float32),
            pltpu.SemaphoreType.DMA,
            pltpu.SemaphoreType.DMA,
        ],
    )
    return kfn(out, src_idx)


def kernel(positions, hidden_states, kv_cache, w_pre, W_qkv, W_o, w_post, W_gate):
    pos2d = positions.astype(jnp.float32).reshape(T, 1)
    wpre2d = w_pre.reshape(1, HIDDEN)
    wpost2d = w_post.reshape(1, HIDDEN)

    q, k, v = pl.pallas_call(
        _qkv_body,
        grid=(T // TB,),
        in_specs=[
            pl.BlockSpec((TB, 1), lambda i: (i, 0)),
            pl.BlockSpec((TB, HIDDEN), lambda i: (i, 0)),
            pl.BlockSpec((1, HIDDEN), lambda i: (0, 0)),
            pl.BlockSpec((Q_SIZE + 2 * KV_SIZE, HIDDEN), lambda i: (0, 0)),
        ],
        out_specs=[
            pl.BlockSpec((TB, Q_SIZE), lambda i: (i, 0)),
            pl.BlockSpec((TB, KV_SIZE), lambda i: (i, 0)),
            pl.BlockSpec((TB, KV_SIZE), lambda i: (i, 0)),
        ],
        out_shape=[
            jax.ShapeDtypeStruct((T, Q_SIZE), jnp.bfloat16),
            jax.ShapeDtypeStruct((T, KV_SIZE), jnp.bfloat16),
            jax.ShapeDtypeStruct((T, KV_SIZE), jnp.bfloat16),
        ],
    )(pos2d, hidden_states, wpre2d, W_qkv)

    attn = pl.pallas_call(
        _attn_body,
        grid=(T // QB,),
        in_specs=[
            pl.BlockSpec((QB, Q_SIZE), lambda qb: (qb, 0)),
            pl.BlockSpec((T, KV_SIZE), lambda qb: (0, 0)),
            pl.BlockSpec((T, KV_SIZE), lambda qb: (0, 0)),
        ],
        out_specs=pl.BlockSpec((QB, Q_SIZE), lambda qb: (qb, 0)),
        out_shape=jax.ShapeDtypeStruct((T, Q_SIZE), jnp.float32),
    )(q, k, v)

    out = pl.pallas_call(
        _oproj_body,
        grid=(T // TB,),
        in_specs=[
            pl.BlockSpec((TB, Q_SIZE), lambda i: (i, 0)),
            pl.BlockSpec((HIDDEN, Q_SIZE), lambda i: (0, 0)),
            pl.BlockSpec((TB, HIDDEN), lambda i: (i, 0)),
            pl.BlockSpec((1, HIDDEN), lambda i: (0, 0)),
        ],
        out_specs=pl.BlockSpec((TB, HIDDEN), lambda i: (i, 0)),
        out_shape=jax.ShapeDtypeStruct((T, HIDDEN), jnp.float32),
    )(attn, W_o, hidden_states, wpost2d)

    key_r = jax.random.fold_in(jax.random.key(0), 123)
    expert_logits = jax.random.uniform(key_r, (T, NUM_EXPERTS), dtype=jnp.float32)
    topk_weights, topk_ids = lax.top_k(expert_logits, TOP_K)
    topk_weights = topk_weights / jnp.sum(topk_weights, axis=-1, keepdims=True)
    flat_ids = topk_ids.reshape(-1)
    reorder_ids = jnp.argsort(flat_ids)
    src_idx = (reorder_ids // TOP_K).astype(jnp.int32)

    permuted_output = _permute_gather_sc(out, src_idx)
    return (permuted_output, topk_weights, topk_ids, reorder_ids)
